# baseline (speedup 1.0000x reference)
import functools

import jax
import jax.numpy as jnp
from jax import lax
from jax.experimental import pallas as pl
from jax.experimental.pallas import tpu as pltpu
from jax.experimental.pallas import tpu_sc as plsc
from jax._src.pallas import mpmd as _mpmd

_NCHUNK = 4


def _body(S, U, rows_per_w, slab, NC, cache_hbm, update_hbm, pos_hbm,
          out_hbm, upd_v, idx_v, pos_v, csem, usem):
    w = lax.axis_index("s") * NC + lax.axis_index("c")
    base = w * rows_per_w
    r0 = w * slab
    chunk = slab // _NCHUNK
    copies = [
        pltpu.async_copy(
            cache_hbm.at[pl.ds(r0 + k * chunk, chunk)],
            out_hbm.at[pl.ds(r0 + k * chunk, chunk)],
            csem,
        )
        for k in range(_NCHUNK)
    ]
    upd_copy = pltpu.async_copy(
        update_hbm.at[pl.ds(base, rows_per_w)], upd_v, usem)
    pltpu.sync_copy(pos_hbm, pos_v)
    posv = pos_v[...]
    lane = lax.iota(jnp.int32, 16)
    ub = U.bit_length() - 1
    sb = S.bit_length() - 1
    rel = ((lane >> ub) << sb) + (posv + (lane & (U - 1))) % S
    for c in range(rows_per_w // 16):
        idx_v[pl.ds(c * 16, 16)] = (((base + c * 16) >> ub) << sb) + rel
    upd_copy.wait()
    for cp in copies:
        cp.wait()
    pltpu.async_copy(upd_v, out_hbm.at[idx_v], usem).wait()


def kernel(cache, update, pos):
    B, H, S, D = cache.shape
    U = update.shape[-2]
    n_rows = B * H * U
    NW = 32
    rows_per_w = n_rows // NW
    slab = (B * H * S) // NW

    cache_flat = cache.reshape(B * H * S, D)
    update_flat = update.reshape(n_rows, D)
    pos_arr = jnp.broadcast_to(jnp.asarray(pos, jnp.int32), (16,))

    mesh = plsc.VectorSubcoreMesh(core_axis_name="c", subcore_axis_name="s")
    NC = 2
    body = functools.partial(_body, S, U, rows_per_w, slab, NC)
    run = _mpmd._mpmd_map(
        [(mesh, body)],
        jax.ShapeDtypeStruct((B * H * S, D), cache.dtype),
        scratch_types=[
            pltpu.VMEM((rows_per_w, D), jnp.float32),
            pltpu.VMEM((rows_per_w,), jnp.int32),
            pltpu.VMEM((16,), jnp.int32),
            pltpu.SemaphoreType.DMA,
            pltpu.SemaphoreType.DMA,
        ],
        name="kvcache_copy_scatter_sc",
    )
    out = run(cache_flat, update_flat, pos_arr)
    return out.reshape(B, H, S, D)


# speedup vs baseline: 43.9678x; 43.9678x over previous
import functools

import jax
import jax.numpy as jnp
from jax import lax
from jax.experimental import pallas as pl
from jax.experimental.pallas import tpu as pltpu
from jax.experimental.pallas import tpu_sc as plsc
from jax._src.pallas import mpmd as _mpmd


def _scatter_body(S, U, rows_per_w, NC, cache_hbm, update_hbm,
                  pos_hbm, out_hbm, upd_v, idx_v, pos_v, usem, psem):
    del cache_hbm
    w = lax.axis_index("s") * NC + lax.axis_index("c")
    base = w * rows_per_w
    upd_copy = pltpu.async_copy(
        update_hbm.at[pl.ds(base, rows_per_w)], upd_v, usem)
    pos_copy = pltpu.async_copy(pos_hbm, pos_v, psem)
    pos_copy.wait()
    posv = pos_v[...]
    lane = lax.iota(jnp.int32, 16)
    ub = U.bit_length() - 1
    sb = S.bit_length() - 1
    rel = ((lane >> ub) << sb) + (posv + (lane & (U - 1))) % S
    for c in range(rows_per_w // 16):
        idx_v[pl.ds(c * 16, 16)] = (((base + c * 16) >> ub) << sb) + rel
    upd_copy.wait()
    pltpu.async_copy(upd_v, out_hbm.at[idx_v], usem).wait()


def kernel(cache, update, pos):
    B, H, S, D = cache.shape
    U = update.shape[-2]
    n_rows = B * H * U
    NW = 32
    rows_per_w = n_rows // NW

    cache_flat = cache.reshape(B * H * S, D)
    update_flat = update.reshape(n_rows, D)
    pos_arr = jnp.broadcast_to(jnp.asarray(pos, jnp.int32), (16,))

    mesh = plsc.VectorSubcoreMesh(core_axis_name="c", subcore_axis_name="s")
    NC = 2
    body = functools.partial(_scatter_body, S, U, rows_per_w, NC)
    run = _mpmd._mpmd_map(
        [(mesh, body)],
        jax.ShapeDtypeStruct((B * H * S, D), cache.dtype),
        input_output_aliases={0: 0},
        scratch_types=[
            pltpu.VMEM((rows_per_w, D), jnp.float32),
            pltpu.VMEM((rows_per_w,), jnp.int32),
            pltpu.VMEM((16,), jnp.int32),
            pltpu.SemaphoreType.DMA,
            pltpu.SemaphoreType.DMA,
        ],
        name="kvcache_scatter_sc",
    )
    out = run(cache_flat, update_flat, pos_arr)
    return out.reshape(B, H, S, D)


# speedup vs baseline: 44.5167x; 1.0125x over previous
import functools

import jax
import jax.numpy as jnp
from jax import lax
from jax.experimental import pallas as pl
from jax.experimental.pallas import tpu as pltpu


def _body(S, U, pos_ref, cache_ref, upd_ref, out_ref):
    out_ref[...] = cache_ref[...]
    p = pos_ref[0]
    for i in range(U):
        r = lax.rem(p + i, S)
        out_ref[0, pl.ds(r, 1), :] = upd_ref[0, pl.ds(i, 1), :]


def kernel(cache, update, pos):
    B, H, S, D = cache.shape
    U = update.shape[-2]
    BH = B * H
    cache3 = cache.reshape(BH, S, D)
    update3 = update.reshape(BH, U, D)
    pos_arr = jnp.asarray(pos, jnp.int32).reshape(1)

    out = pl.pallas_call(
        functools.partial(_body, S, U),
        out_shape=jax.ShapeDtypeStruct((BH, S, D), cache.dtype),
        grid=(BH,),
        in_specs=[
            pl.BlockSpec(memory_space=pltpu.SMEM),
            pl.BlockSpec((1, S, D), lambda i: (i, 0, 0)),
            pl.BlockSpec((1, U, D), lambda i: (i, 0, 0)),
        ],
        out_specs=pl.BlockSpec((1, S, D), lambda i: (i, 0, 0)),
        compiler_params=pltpu.CompilerParams(
            dimension_semantics=("arbitrary",),
        ),
        name="kvcache_fused_copy_update",
    )(pos_arr, cache3, update3)
    return out.reshape(B, H, S, D)
